# baseline (speedup 1.0000x reference)
import functools

import jax
import jax.numpy as jnp
from jax import lax
from jax.experimental import pallas as pl
from jax.experimental.pallas import tpu as pltpu
from jax.experimental.pallas import tpu_sc as plsc

NUM_CAT = 26
VOCAB = 100000
EMB = 128
BATCH = 16384

_NC = 2
_NS = 16
_NW = _NC * _NS
_BPW = BATCH // _NW
_C = 128
_G = 128
_L = 16
_RING = 4
_NHALF = _BPW // _C
_NCHUNK = NUM_CAT * _NHALF

_MB = 1024


def _mlp_body(x_ref, w1_ref, b1_ref, w2_ref, b2_ref, w3_ref, b3_ref,
              mlp_ref):
    h = jnp.dot(x_ref[...], w1_ref[...], preferred_element_type=jnp.float32)
    h = jnp.maximum(h + b1_ref[...], 0.0)
    h = jnp.dot(h, w2_ref[...], preferred_element_type=jnp.float32)
    h = jnp.maximum(h + b2_ref[...], 0.0)
    h = jnp.dot(h, w3_ref[...], preferred_element_type=jnp.float32)
    h = jnp.maximum(h + b3_ref[...], 0.0)
    mlp_ref[...] = h


def _mlp(x, W1, b1, W2, b2, W3, b3):
    grid = (BATCH // _MB,)
    full = lambda *s: pl.BlockSpec(s, lambda i: (0,) * len(s))
    return pl.pallas_call(
        _mlp_body,
        grid=grid,
        in_specs=[
            pl.BlockSpec((_MB, 13), lambda i: (i, 0)),
            full(13, 512), full(1, 512),
            full(512, 256), full(1, 256),
            full(256, EMB), full(1, EMB),
        ],
        out_specs=pl.BlockSpec((_MB, EMB), lambda i: (i, 0)),
        out_shape=jax.ShapeDtypeStruct((BATCH, EMB), jnp.float32),
    )(x, W1, b1.reshape(1, 512), W2, b2.reshape(1, 256),
      W3, b3.reshape(1, EMB))


def _plane0_body(mlp_ref, out_in_ref, out_ref):
    del out_in_ref
    out_ref[...] = mlp_ref[...][None]


_WB = 2048


def _write_plane0(mlp, out_fm):
    grid = (BATCH // _WB,)
    return pl.pallas_call(
        _plane0_body,
        grid=grid,
        in_specs=[
            pl.BlockSpec((_WB, EMB), lambda i: (i, 0)),
            pl.BlockSpec((1, 8, EMB), lambda i: (0, 0, 0)),
        ],
        out_specs=pl.BlockSpec((1, _WB, EMB), lambda i: (0, i, 0)),
        out_shape=jax.ShapeDtypeStruct((1 + NUM_CAT, BATCH, EMB),
                                       jnp.float32),
        input_output_aliases={1: 0},
    )(mlp, out_fm)


def _sc_gather_body(table_hbm, cat_hbm, out_hbm, idx_v, *rest):
    bufs, sems = rest[:_RING], rest[_RING:]
    slots = tuple((bufs[s], sems[2 * s], sems[2 * s + 1])
                  for s in range(_RING))
    wid = lax.axis_index("s") * _NC + lax.axis_index("c")
    base = wid * _BPW

    pltpu.sync_copy(cat_hbm.at[:, pl.ds(base, _BPW)], idx_v)

    def fire_fetch(t, slot):
        buf_v, gsem, _ = slot
        f = t // _NHALF
        half = t % _NHALF
        c0 = half * _C
        off = f * VOCAB
        for k in range(_C // _L):
            sl = pl.ds(c0 + k * _L, _L)
            idx_v[f, sl] = idx_v[f, sl] + off
        for j in range(_C // _G):
            pltpu.async_copy(
                table_hbm.at[idx_v.at[f, pl.ds(c0 + j * _G, _G)]],
                buf_v.at[pl.ds(j * _G, _G)], gsem)

    def drain_fetch(slot):
        buf_v, gsem, _ = slot
        pltpu.make_async_copy(table_hbm.at[pl.ds(0, _C)], buf_v,
                              gsem).wait()

    def _out_slice(t):
        f = t // _NHALF
        half = t % _NHALF
        return out_hbm.at[1 + f, pl.ds(base + half * _C, _C)]

    def fire_out(t, slot):
        buf_v, _, osem = slot
        pltpu.async_copy(buf_v, _out_slice(t), osem)

    def wait_out(t, slot):
        buf_v, _, osem = slot
        pltpu.make_async_copy(buf_v, _out_slice(t), osem).wait()

    for s in range(_RING):
        fire_fetch(s, slots[s])

    def steady(i, carry):
        tr = _RING * i
        for s in range(_RING):
            drain_fetch(slots[s])
            fire_out(tr + s, slots[s])
            wait_out(tr + s, slots[s])
            fire_fetch(tr + s + _RING, slots[s])
        return carry

    nsteady = (_NCHUNK - _RING) // _RING
    lax.fori_loop(0, nsteady, steady, 0)

    inflight = [(_RING * nsteady + s, s) for s in range(_RING)]
    unfetched = list(range(_RING * nsteady + _RING, _NCHUNK))
    pending = []
    while inflight:
        t, s = inflight.pop(0)
        drain_fetch(slots[s])
        fire_out(t, slots[s])
        if unfetched:
            wait_out(t, slots[s])
            t2 = unfetched.pop(0)
            fire_fetch(t2, slots[s])
            inflight.append((t2, s))
        else:
            pending.append((t, s))
    for t, s in pending:
        wait_out(t, slots[s])


_sc_gather = functools.partial(
    pl.kernel,
    out_type=jax.ShapeDtypeStruct((1 + NUM_CAT, BATCH, EMB), jnp.float32),
    mesh=plsc.VectorSubcoreMesh(core_axis_name="c", subcore_axis_name="s",
                                num_cores=_NC, num_subcores=_NS),
    scratch_types=(
        [pltpu.VMEM((NUM_CAT, _BPW), jnp.int32)]
        + [pltpu.VMEM((_C, EMB), jnp.float32)] * _RING
        + [pltpu.SemaphoreType.DMA] * (2 * _RING)
    ),
)(_sc_gather_body)


def kernel(numerical_input, categorical_inputs, table, W1, b1, W2, b2, W3,
           b3, offsets):
    del offsets
    out_fm = _sc_gather(table, categorical_inputs.T)
    mlp_out = _mlp(numerical_input, W1, b1, W2, b2, W3, b3)
    out_fm = _write_plane0(mlp_out, out_fm)
    return jnp.transpose(out_fm, (1, 0, 2))


# speedup vs baseline: 1.0022x; 1.0022x over previous
import functools

import jax
import jax.numpy as jnp
from jax import lax
from jax.experimental import pallas as pl
from jax.experimental.pallas import tpu as pltpu
from jax.experimental.pallas import tpu_sc as plsc

NUM_CAT = 26
VOCAB = 100000
EMB = 128
BATCH = 16384

_NC = 2
_NS = 16
_NW = _NC * _NS
_BPW = BATCH // _NW
_C = 128
_G = 128
_L = 16
_RING = 6
_NHALF = _BPW // _C
_NCHUNK = NUM_CAT * _NHALF

_MB = 1024


def _mlp_body(x_ref, w1_ref, b1_ref, w2_ref, b2_ref, w3_ref, b3_ref,
              mlp_ref):
    h = jnp.dot(x_ref[...], w1_ref[...], preferred_element_type=jnp.float32)
    h = jnp.maximum(h + b1_ref[...], 0.0)
    h = jnp.dot(h, w2_ref[...], preferred_element_type=jnp.float32)
    h = jnp.maximum(h + b2_ref[...], 0.0)
    h = jnp.dot(h, w3_ref[...], preferred_element_type=jnp.float32)
    h = jnp.maximum(h + b3_ref[...], 0.0)
    mlp_ref[...] = h


def _mlp(x, W1, b1, W2, b2, W3, b3):
    grid = (BATCH // _MB,)
    full = lambda *s: pl.BlockSpec(s, lambda i: (0,) * len(s))
    return pl.pallas_call(
        _mlp_body,
        grid=grid,
        in_specs=[
            pl.BlockSpec((_MB, 13), lambda i: (i, 0)),
            full(13, 512), full(1, 512),
            full(512, 256), full(1, 256),
            full(256, EMB), full(1, EMB),
        ],
        out_specs=pl.BlockSpec((_MB, EMB), lambda i: (i, 0)),
        out_shape=jax.ShapeDtypeStruct((BATCH, EMB), jnp.float32),
    )(x, W1, b1.reshape(1, 512), W2, b2.reshape(1, 256),
      W3, b3.reshape(1, EMB))


def _plane0_body(mlp_ref, out_in_ref, out_ref):
    del out_in_ref
    out_ref[...] = mlp_ref[...][None]


_WB = 2048


def _write_plane0(mlp, out_fm):
    grid = (BATCH // _WB,)
    return pl.pallas_call(
        _plane0_body,
        grid=grid,
        in_specs=[
            pl.BlockSpec((_WB, EMB), lambda i: (i, 0)),
            pl.BlockSpec((1, 8, EMB), lambda i: (0, 0, 0)),
        ],
        out_specs=pl.BlockSpec((1, _WB, EMB), lambda i: (0, i, 0)),
        out_shape=jax.ShapeDtypeStruct((1 + NUM_CAT, BATCH, EMB),
                                       jnp.float32),
        input_output_aliases={1: 0},
    )(mlp, out_fm)


def _sc_gather_body(table_hbm, cat_hbm, out_hbm, idx_v, *rest):
    bufs, sems = rest[:_RING], rest[_RING:]
    slots = tuple((bufs[s], sems[2 * s], sems[2 * s + 1])
                  for s in range(_RING))
    wid = lax.axis_index("s") * _NC + lax.axis_index("c")
    base = wid * _BPW

    pltpu.sync_copy(cat_hbm.at[:, pl.ds(base, _BPW)], idx_v)

    def fire_fetch(t, slot):
        buf_v, gsem, _ = slot
        f = t // _NHALF
        half = t % _NHALF
        c0 = half * _C
        off = f * VOCAB
        for k in range(_C // _L):
            sl = pl.ds(c0 + k * _L, _L)
            idx_v[f, sl] = idx_v[f, sl] + off
        for j in range(_C // _G):
            pltpu.async_copy(
                table_hbm.at[idx_v.at[f, pl.ds(c0 + j * _G, _G)]],
                buf_v.at[pl.ds(j * _G, _G)], gsem)

    def drain_fetch(slot):
        buf_v, gsem, _ = slot
        pltpu.make_async_copy(table_hbm.at[pl.ds(0, _C)], buf_v,
                              gsem).wait()

    def _out_slice(t):
        f = t // _NHALF
        half = t % _NHALF
        return out_hbm.at[1 + f, pl.ds(base + half * _C, _C)]

    def fire_out(t, slot):
        buf_v, _, osem = slot
        pltpu.async_copy(buf_v, _out_slice(t), osem)

    def wait_out(t, slot):
        buf_v, _, osem = slot
        pltpu.make_async_copy(buf_v, _out_slice(t), osem).wait()

    for s in range(_RING):
        fire_fetch(s, slots[s])

    def steady(i, carry):
        tr = _RING * i
        for s in range(_RING):
            drain_fetch(slots[s])
            fire_out(tr + s, slots[s])
            wait_out(tr + s, slots[s])
            fire_fetch(tr + s + _RING, slots[s])
        return carry

    nsteady = (_NCHUNK - _RING) // _RING
    lax.fori_loop(0, nsteady, steady, 0)

    inflight = [(_RING * nsteady + s, s) for s in range(_RING)]
    unfetched = list(range(_RING * nsteady + _RING, _NCHUNK))
    pending = []
    while inflight:
        t, s = inflight.pop(0)
        drain_fetch(slots[s])
        fire_out(t, slots[s])
        if unfetched:
            wait_out(t, slots[s])
            t2 = unfetched.pop(0)
            fire_fetch(t2, slots[s])
            inflight.append((t2, s))
        else:
            pending.append((t, s))
    for t, s in pending:
        wait_out(t, slots[s])


_sc_gather = functools.partial(
    pl.kernel,
    out_type=jax.ShapeDtypeStruct((1 + NUM_CAT, BATCH, EMB), jnp.float32),
    mesh=plsc.VectorSubcoreMesh(core_axis_name="c", subcore_axis_name="s",
                                num_cores=_NC, num_subcores=_NS),
    scratch_types=(
        [pltpu.VMEM((NUM_CAT, _BPW), jnp.int32)]
        + [pltpu.VMEM((_C, EMB), jnp.float32)] * _RING
        + [pltpu.SemaphoreType.DMA] * (2 * _RING)
    ),
)(_sc_gather_body)


def kernel(numerical_input, categorical_inputs, table, W1, b1, W2, b2, W3,
           b3, offsets):
    del offsets
    out_fm = _sc_gather(table, categorical_inputs.T)
    mlp_out = _mlp(numerical_input, W1, b1, W2, b2, W3, b3)
    out_fm = _write_plane0(mlp_out, out_fm)
    return jnp.transpose(out_fm, (1, 0, 2))
